# baseline (speedup 1.0000x reference)
import functools

import jax
import jax.numpy as jnp
from jax import lax
from jax.experimental import pallas as pl
from jax.experimental.pallas import tpu as pltpu
from jax.experimental.pallas import tpu_sc as plsc

N = 320000
D = 128
NSYS = 10000
NC = 2
NS = 16
CHUNK = 80
NCHUNKS = N // CHUNK
CHUNKS_PER_SC = NCHUNKS // NC
NBUF = 4
K_ITERS = ((CHUNKS_PER_SC + NS - 1) // NS + NBUF - 1) // NBUF * NBUF
ACC_ROWS = NSYS + 8
WB = 40
NWB = NSYS // WB


def _sc_partial_sums(features, batch_index):
    mesh = plsc.VectorSubcoreMesh(core_axis_name="c", subcore_axis_name="s")

    @functools.partial(
        pl.kernel,
        out_type=jax.ShapeDtypeStruct((NC, NSYS, D), jnp.float32),
        mesh=mesh,
        scratch_types=[
            pltpu.VMEM((NBUF, CHUNK, D), jnp.float32),
            pltpu.VMEM((NBUF, CHUNK), jnp.int32),
            pltpu.VMEM((WB, D), jnp.float32),
            pltpu.VMEM_SHARED((ACC_ROWS, D), jnp.float32),
            pltpu.SemaphoreType.DMA,
            pltpu.SemaphoreType.DMA,
            pltpu.SemaphoreType.DMA,
            pltpu.SemaphoreType.DMA,
            pltpu.SemaphoreType.DMA,
            pltpu.SemaphoreType.DMA,
            pltpu.SemaphoreType.DMA,
            pltpu.SemaphoreType.DMA,
        ],
    )
    def body(feat_hbm, idx_hbm, out_hbm, row_v, idx_v, zero_v, acc,
             g0, g1, g2, g3, s0, s1, s2, s3):
        c = lax.axis_index("c")
        t = lax.axis_index("s")

        def zrow(i, _):
            for k in range(D // 16):
                zero_v[i, pl.ds(16 * k, 16)] = jnp.zeros((16,), jnp.float32)
            return 0

        lax.fori_loop(0, WB, zrow, 0)

        def zchunk(i, _):
            j = t + NS * i

            @pl.when(j < NWB)
            def _():
                pltpu.sync_copy(zero_v, acc.at[pl.ds(WB * j, WB)])

            return 0

        lax.fori_loop(0, (NWB + NS - 1) // NS, zchunk, 0)
        plsc.subcore_barrier()

        base = CHUNKS_PER_SC * c
        gsem = (g0, g1, g2, g3)
        ssem = (s0, s1, s2, s3)

        def rstart(i):
            jj = t + NS * i
            jc = jnp.minimum(jj, CHUNKS_PER_SC - 1)
            return CHUNK * (base + jc)

        def start_gather(i, p):
            r = rstart(i)
            pltpu.async_copy(idx_hbm.at[pl.ds(r, CHUNK)], idx_v.at[p], gsem[p])
            pltpu.async_copy(feat_hbm.at[pl.ds(r, CHUNK)], row_v.at[p], gsem[p])

        def wait_gather(i, p):
            r = rstart(i)
            pltpu.make_async_copy(
                idx_hbm.at[pl.ds(r, CHUNK)], idx_v.at[p], gsem[p]
            ).wait()
            pltpu.make_async_copy(
                feat_hbm.at[pl.ds(r, CHUNK)], row_v.at[p], gsem[p]
            ).wait()

        def step(i, p, first, last):
            wait_gather(i, p)
            m = (t + NS * i < CHUNKS_PER_SC).astype(jnp.int32)
            fill = NSYS * (1 - m)
            for k in range(CHUNK // 16):
                v = idx_v[p, pl.ds(16 * k, 16)]
                idx_v[p, pl.ds(16 * k, 16)] = v * m + fill
            pltpu.async_copy(row_v.at[p], acc.at[idx_v.at[p]], ssem[p], add=True)
            if not last:
                g = (p + 2) % NBUF
                if not first:
                    pltpu.make_async_copy(
                        row_v.at[g], acc.at[idx_v.at[g]], ssem[g]
                    ).wait()
                start_gather(i + 2, g)

        start_gather(0, 0)
        start_gather(1, 1)
        for i in range(NBUF):
            step(i, i, first=(i < 2), last=False)

        def aloop(i4, _):
            i = NBUF * i4
            for p in range(NBUF):
                step(i + p, p, first=False, last=False)
            return 0

        lax.fori_loop(1, K_ITERS // NBUF - 1, aloop, 0)

        for u in range(NBUF):
            i = K_ITERS - NBUF + u
            step(i, u, first=False, last=(u >= 2))

        for p in range(NBUF):
            pltpu.make_async_copy(
                row_v.at[p], acc.at[idx_v.at[p]], ssem[p]
            ).wait()

        plsc.subcore_barrier()

        def wchunk(i, _):
            j = t + NS * i

            @pl.when(j < NWB)
            def _():
                pltpu.sync_copy(
                    acc.at[pl.ds(WB * j, WB)], out_hbm.at[c, pl.ds(WB * j, WB)]
                )

            return 0

        lax.fori_loop(0, (NWB + NS - 1) // NS, wchunk, 0)

    return body(features, batch_index)


def _tc_combine(partials):
    BS = 1000

    def body(p_ref, o_ref):
        o_ref[...] = p_ref[0] + p_ref[1]

    return pl.pallas_call(
        body,
        out_shape=jax.ShapeDtypeStruct((NSYS, D), jnp.float32),
        grid=(NSYS // BS,),
        in_specs=[pl.BlockSpec((NC, BS, D), lambda i: (0, i, 0))],
        out_specs=pl.BlockSpec((BS, D), lambda i: (i, 0)),
    )(partials)


def kernel(features, batch_index, natoms):
    del natoms
    bi = batch_index.astype(jnp.int32)
    partials = _sc_partial_sums(features, bi)
    return _tc_combine(partials)


# speedup vs baseline: 1.1336x; 1.1336x over previous
import functools

import jax
import jax.numpy as jnp
from jax import lax
from jax.experimental import pallas as pl
from jax.experimental.pallas import tpu as pltpu
from jax.experimental.pallas import tpu_sc as plsc

N = 320000
D = 128
NSYS = 10000
NC = 2
NS = 16
CHUNK = 128
NCHUNKS = N // CHUNK
CHUNKS_PER_SC = NCHUNKS // NC
PER_TILE = (CHUNKS_PER_SC // NS) & ~1
SEG_PER_SC = NSYS // NC
DUMMY = SEG_PER_SC
ACC_ROWS = SEG_PER_SC + 8
WB = 40
NWB = SEG_PER_SC // WB


def _sc_segment_sum(features, batch_index):
    mesh = plsc.VectorSubcoreMesh(core_axis_name="c", subcore_axis_name="s")

    @functools.partial(
        pl.kernel,
        out_type=jax.ShapeDtypeStruct((NSYS, D), jnp.float32),
        mesh=mesh,
        scratch_types=[
            pltpu.VMEM((2, CHUNK, D), jnp.float32),
            pltpu.VMEM((2, CHUNK), jnp.int32),
            pltpu.VMEM((WB, D), jnp.float32),
            pltpu.VMEM_SHARED((ACC_ROWS, D), jnp.float32),
            pltpu.SemaphoreType.DMA,
            pltpu.SemaphoreType.DMA,
        ],
    )
    def body(feat_hbm, idx_hbm, out_hbm, row_v, idx_v, zero_v, acc, sem0, sem1):
        c = lax.axis_index("c")
        t = lax.axis_index("s")
        segbase = SEG_PER_SC * c

        dummy_vec = jnp.full((16,), DUMMY, jnp.int32)

        def localize_idx(p):
            for k in range(CHUNK // 16):
                v = idx_v[p, pl.ds(16 * k, 16)] - segbase
                m = (v >= 0) & (v < SEG_PER_SC)
                idx_v[p, pl.ds(16 * k, 16)] = jnp.where(m, v, dummy_vec)

        def zrow(i, _):
            for k in range(D // 16):
                zero_v[i, pl.ds(16 * k, 16)] = jnp.zeros((16,), jnp.float32)
            return 0

        lax.fori_loop(0, WB, zrow, 0)

        def zchunk(i, _):
            j = t + NS * i

            @pl.when(j < NWB)
            def _():
                pltpu.sync_copy(zero_v, acc.at[pl.ds(WB * j, WB)])

            return 0

        lax.fori_loop(0, (NWB + NS - 1) // NS, zchunk, 0)
        plsc.subcore_barrier()

        base = CHUNKS_PER_SC * c
        sems = (sem0, sem1)

        def start_gather(i, p):
            r = CHUNK * (base + t + NS * i)
            pltpu.async_copy(idx_hbm.at[pl.ds(r, CHUNK)], idx_v.at[p], sems[p])
            pltpu.async_copy(feat_hbm.at[pl.ds(r, CHUNK)], row_v.at[p], sems[p])

        def wait_gather(i, p):
            r = CHUNK * (base + t + NS * i)
            pltpu.make_async_copy(
                idx_hbm.at[pl.ds(r, CHUNK)], idx_v.at[p], sems[p]
            ).wait()
            pltpu.make_async_copy(
                feat_hbm.at[pl.ds(r, CHUNK)], row_v.at[p], sems[p]
            ).wait()

        start_gather(0, 0)
        start_gather(1, 1)

        def achunk(i2, _):
            for p in range(2):
                i = 2 * i2 + p
                wait_gather(i, p)
                localize_idx(p)
                pltpu.sync_copy(row_v.at[p], acc.at[idx_v.at[p]], add=True)

                @pl.when(i + 2 < PER_TILE)
                def _():
                    start_gather(i + 2, p)

            return 0

        lax.fori_loop(0, PER_TILE // 2, achunk, 0)

        @pl.when(t < CHUNKS_PER_SC - NS * PER_TILE)
        def _():
            r = CHUNK * (base + NS * PER_TILE + t)
            pltpu.sync_copy(idx_hbm.at[pl.ds(r, CHUNK)], idx_v.at[0])
            pltpu.sync_copy(feat_hbm.at[pl.ds(r, CHUNK)], row_v.at[0])
            localize_idx(0)
            pltpu.sync_copy(row_v.at[0], acc.at[idx_v.at[0]], add=True)

        k0 = (1 - c) * (CHUNKS_PER_SC + t) + c * (CHUNKS_PER_SC - 1 - t)
        kstep = NS - 2 * NS * c
        max_sweep = (CHUNKS_PER_SC + NS - 1) // NS

        def sweep_body(i, cont):
            k = k0 + kstep * i
            kc = jnp.clip(k, 0, NCHUNKS - 1)
            ok = (cont == 1) & (k >= 0) & (k < NCHUNKS)

            @pl.when(ok)
            def _():
                pltpu.sync_copy(
                    idx_hbm.at[pl.ds(CHUNK * kc, CHUNK)], idx_v.at[0]
                )

            mn = idx_v[0, pl.ds(0, 16)][0]
            mx = idx_v[0, pl.ds(CHUNK - 16, 16)][15]
            a = (mn < SEG_PER_SC).astype(jnp.int32)
            b = (mx >= SEG_PER_SC).astype(jnp.int32)
            has_own = (1 - c) * a + c * b
            proceed = ok & (has_own == 1)

            @pl.when(proceed)
            def _():
                pltpu.sync_copy(
                    feat_hbm.at[pl.ds(CHUNK * kc, CHUNK)], row_v.at[0]
                )
                localize_idx(0)
                pltpu.sync_copy(row_v.at[0], acc.at[idx_v.at[0]], add=True)

            return proceed.astype(jnp.int32)

        lax.fori_loop(0, max_sweep, sweep_body, jnp.int32(1))
        plsc.subcore_barrier()

        def wchunk(i, _):
            j = t + NS * i

            @pl.when(j < NWB)
            def _():
                pltpu.sync_copy(
                    acc.at[pl.ds(WB * j, WB)],
                    out_hbm.at[pl.ds(SEG_PER_SC * c + WB * j, WB)],
                )

            return 0

        lax.fori_loop(0, (NWB + NS - 1) // NS, wchunk, 0)

    return body(features, batch_index)


def kernel(features, batch_index, natoms):
    del natoms
    bi = batch_index.astype(jnp.int32)
    return _sc_segment_sum(features, bi)
